# baseline (speedup 1.0000x reference)
import functools

import jax
import jax.numpy as jnp
from jax import lax
from jax.experimental import pallas as pl
from jax.experimental.pallas import tpu as pltpu
from jax.experimental.pallas import tpu_sc as plsc

NC = 2
NS = 16
NW = NC * NS
C = 64
D = 256


def _sc_decode(z, sidx, didx, *, chunks, epw, e_pad):
    n_nodes = z.shape[0]
    mesh = plsc.VectorSubcoreMesh(
        core_axis_name="c", subcore_axis_name="s", num_cores=NC, num_subcores=NS
    )

    @functools.partial(
        pl.kernel,
        out_type=jax.ShapeDtypeStruct((e_pad,), jnp.float32),
        mesh=mesh,
        compiler_params=pltpu.CompilerParams(
            use_tc_tiling_on_sc=False, needs_layout_passes=False
        ),
        scratch_types=[
            pltpu.VMEM((chunks, C), jnp.int32),
            pltpu.VMEM((chunks, C), jnp.int32),
            pltpu.VMEM((C, D), jnp.bfloat16),
            pltpu.VMEM((C, D), jnp.bfloat16),
            pltpu.VMEM((C, D), jnp.bfloat16),
            pltpu.VMEM((C, D), jnp.bfloat16),
            pltpu.VMEM((epw,), jnp.float32),
            pltpu.VMEM((C, 17), jnp.float32),
            pltpu.VMEM_SHARED((n_nodes, D), jnp.bfloat16),
            pltpu.SemaphoreType.DMA,
            pltpu.SemaphoreType.DMA,
        ],
    )
    def k(z_hbm, sidx_hbm, didx_hbm, out_hbm,
          sidx_v, didx_v, sb0, sb1, db0, db1, out_v, tr, z_sh, sem0, sem1):
        cid = lax.axis_index("c")
        sid = lax.axis_index("s")
        wid = sid * NC + cid

        band = n_nodes // NS
        pltpu.sync_copy(z_hbm.at[pl.ds(sid * band, band)],
                        z_sh.at[pl.ds(sid * band, band)])
        pltpu.sync_copy(sidx_hbm.at[wid], sidx_v)
        pltpu.sync_copy(didx_hbm.at[wid], didx_v)
        plsc.subcore_barrier()

        sbufs = (sb0, sb1)
        dbufs = (db0, db1)
        sems = (sem0, sem1)

        H = C // 2

        def start(j, b):
            pltpu.async_copy(z_sh.at[sidx_v.at[j]], sbufs[b], sems[b])
            pltpu.async_copy(z_sh.at[didx_v.at[j]], dbufs[b], sems[b])

        def wait(b):
            pltpu.make_async_copy(z_sh.at[sidx_v.at[0]], sbufs[b], sems[b]).wait()
            pltpu.make_async_copy(z_sh.at[didx_v.at[0]], dbufs[b], sems[b]).wait()

        lanes = lax.iota(jnp.int32, 16)

        def compute(jj, b):
            sb, db = sbufs[b], dbufs[b]

            @plsc.parallel_loop(0, C // 16, 1, unroll=2)
            def gbody(g):
                for ii in range(16):
                    e = g * 16 + ii
                    a0 = jnp.zeros((16,), jnp.float32)
                    a1 = jnp.zeros((16,), jnp.float32)
                    for kk in range(D // 32):
                        p = sb[e, pl.ds(kk * 32, 32)] * db[e, pl.ds(kk * 32, 32)]
                        p0, p1 = plsc.unpack(
                            p, format=plsc.PackFormat.INTERLEAVED,
                            preferred_element_type=jnp.float32)
                        a0 = a0 + p0
                        a1 = a1 + p1
                    tr[e, pl.ds(0, 16)] = a0 + a1
                rows = g * 16 + lanes
                cols = [plsc.load_gather(tr, [rows, jnp.full((16,), dd, jnp.int32)])
                        for dd in range(16)]
                while len(cols) > 1:
                    cols = [x + y for x, y in zip(cols[::2], cols[1::2])]
                out_v[pl.ds(jj * C + g * 16, 16)] = 1.0 / (1.0 + jnp.exp(-cols[0]))

        start(0, 0)

        def pair(p, carry):
            j = p * 2
            for b in range(2):
                jj = j + b
                nxt = jj + 1

                @pl.when(nxt < chunks)
                def _():
                    start(nxt, (b + 1) % 2)

                wait(b)
                compute(jj, b)
            return carry

        lax.fori_loop(0, chunks // 2, pair, 0)
        pltpu.sync_copy(out_v, out_hbm.at[pl.ds(wid * epw, epw)])

    return k(z, sidx, didx)


def kernel(z, edge_index):
    e = edge_index.shape[1]
    per_pair = NW * C
    chunks = -(-e // per_pair)
    chunks += chunks % 2
    epw = chunks * C
    e_pad = NW * epw
    sidx = jnp.pad(edge_index[0], (0, e_pad - e)).reshape(NW, chunks, C)
    didx = jnp.pad(edge_index[1], (0, e_pad - e)).reshape(NW, chunks, C)
    zb = z.astype(jnp.bfloat16)
    out = _sc_decode(zb, sidx, didx, chunks=chunks, epw=epw, e_pad=e_pad)
    return out[:e]


# speedup vs baseline: 1.1606x; 1.1606x over previous
import functools

import jax
import jax.numpy as jnp
from jax import lax
from jax.experimental import pallas as pl
from jax.experimental.pallas import tpu as pltpu
from jax.experimental.pallas import tpu_sc as plsc

NC = 2
NS = 16
NW = NC * NS
C = 64
D = 256


def _sc_decode(z, sidx, didx, *, chunks, epw, e_pad):
    n_nodes = z.shape[0]
    mesh = plsc.VectorSubcoreMesh(
        core_axis_name="c", subcore_axis_name="s", num_cores=NC, num_subcores=NS
    )

    @functools.partial(
        pl.kernel,
        out_type=jax.ShapeDtypeStruct((e_pad,), jnp.float32),
        mesh=mesh,
        compiler_params=pltpu.CompilerParams(
            use_tc_tiling_on_sc=False, needs_layout_passes=False
        ),
        scratch_types=[
            pltpu.VMEM((chunks, C), jnp.int32),
            pltpu.VMEM((chunks, C), jnp.int32),
            pltpu.VMEM((C, D), jnp.bfloat16),
            pltpu.VMEM((C, D), jnp.bfloat16),
            pltpu.VMEM((C, D), jnp.bfloat16),
            pltpu.VMEM((C, D), jnp.bfloat16),
            pltpu.VMEM((epw,), jnp.float32),
            pltpu.VMEM((C, 17), jnp.float32),
            pltpu.VMEM_SHARED((n_nodes, D), jnp.bfloat16),
            pltpu.SemaphoreType.DMA,
            pltpu.SemaphoreType.DMA,
        ],
    )
    def k(z_hbm, sidx_hbm, didx_hbm, out_hbm,
          sidx_v, didx_v, sb0, sb1, db0, db1, out_v, tr, z_sh, sem0, sem1):
        cid = lax.axis_index("c")
        sid = lax.axis_index("s")
        wid = sid * NC + cid

        band = n_nodes // NS
        pltpu.sync_copy(z_hbm.at[pl.ds(sid * band, band)],
                        z_sh.at[pl.ds(sid * band, band)])
        pltpu.sync_copy(sidx_hbm.at[wid], sidx_v)
        pltpu.sync_copy(didx_hbm.at[wid], didx_v)
        plsc.subcore_barrier()

        sbufs = (sb0, sb1)
        dbufs = (db0, db1)
        sems = (sem0, sem1)

        H = C // 2

        def start(j, b):
            pltpu.async_copy(z_sh.at[sidx_v.at[j]], sbufs[b], sems[b])
            pltpu.async_copy(z_sh.at[didx_v.at[j]], dbufs[b], sems[b])

        def wait(b):
            pltpu.make_async_copy(z_sh.at[sidx_v.at[0]], sbufs[b], sems[b]).wait()
            pltpu.make_async_copy(z_sh.at[didx_v.at[0]], dbufs[b], sems[b]).wait()

        lanes = lax.iota(jnp.int32, 16)

        def compute(jj, b):
            sb, db = sbufs[b], dbufs[b]

            def gbody(g, carry):
                for ii in range(16):
                    e = g * 16 + ii
                    a0 = jnp.zeros((16,), jnp.float32)
                    a1 = jnp.zeros((16,), jnp.float32)
                    for kk in range(D // 32):
                        p = sb[e, pl.ds(kk * 32, 32)] * db[e, pl.ds(kk * 32, 32)]
                        p0, p1 = plsc.unpack(
                            p, format=plsc.PackFormat.INTERLEAVED,
                            preferred_element_type=jnp.float32)
                        a0 = a0 + p0
                        a1 = a1 + p1
                    tr[e, pl.ds(0, 16)] = a0 + a1
                rows = g * 16 + lanes
                cols = [plsc.load_gather(tr, [rows, jnp.full((16,), dd, jnp.int32)])
                        for dd in range(16)]
                while len(cols) > 1:
                    cols = [x + y for x, y in zip(cols[::2], cols[1::2])]
                out_v[pl.ds(jj * C + g * 16, 16)] = cols[0]
                return carry

            lax.fori_loop(0, C // 16, gbody, 0)

        start(0, 0)

        def pair(p, carry):
            j = p * 2
            for b in range(2):
                jj = j + b
                nxt = jj + 1

                @pl.when(nxt < chunks)
                def _():
                    start(nxt, (b + 1) % 2)

                wait(b)
                compute(jj, b)
            return carry

        lax.fori_loop(0, chunks // 2, pair, 0)

        def sig(i, carry):
            for u in range(2):
                o = (i * 2 + u) * 16
                x = out_v[pl.ds(o, 16)]
                out_v[pl.ds(o, 16)] = 1.0 / (1.0 + jnp.exp(-x))
            return carry

        lax.fori_loop(0, epw // 32, sig, 0)
        pltpu.sync_copy(out_v, out_hbm.at[pl.ds(wid * epw, epw)])

    return k(z, sidx, didx)


def kernel(z, edge_index):
    e = edge_index.shape[1]
    per_pair = NW * C
    chunks = -(-e // per_pair)
    chunks += chunks % 2
    epw = chunks * C
    e_pad = NW * epw
    sidx = jnp.pad(edge_index[0], (0, e_pad - e)).reshape(NW, chunks, C)
    didx = jnp.pad(edge_index[1], (0, e_pad - e)).reshape(NW, chunks, C)
    zb = z.astype(jnp.bfloat16)
    out = _sc_decode(zb, sidx, didx, chunks=chunks, epw=epw, e_pad=e_pad)
    return out[:e]


# speedup vs baseline: 1.1629x; 1.0019x over previous
import functools

import jax
import jax.numpy as jnp
from jax import lax
from jax.experimental import pallas as pl
from jax.experimental.pallas import tpu as pltpu
from jax.experimental.pallas import tpu_sc as plsc

NC = 2
NS = 16
NW = NC * NS
C = 64
D = 256


def _sc_decode(z, sidx, didx, *, chunks, epw, e_pad):
    n_nodes = z.shape[0]
    mesh = plsc.VectorSubcoreMesh(
        core_axis_name="c", subcore_axis_name="s", num_cores=NC, num_subcores=NS
    )

    @functools.partial(
        pl.kernel,
        out_type=jax.ShapeDtypeStruct((e_pad,), jnp.float32),
        mesh=mesh,
        compiler_params=pltpu.CompilerParams(
            use_tc_tiling_on_sc=False, needs_layout_passes=False
        ),
        scratch_types=[
            pltpu.VMEM((chunks, C), jnp.int32),
            pltpu.VMEM((chunks, C), jnp.int32),
            pltpu.VMEM((C, D), jnp.bfloat16),
            pltpu.VMEM((C, D), jnp.bfloat16),
            pltpu.VMEM((C, D), jnp.bfloat16),
            pltpu.VMEM((C, D), jnp.bfloat16),
            pltpu.VMEM((epw,), jnp.float32),
            pltpu.VMEM((C, 17), jnp.float32),
            pltpu.VMEM_SHARED((n_nodes, D), jnp.bfloat16),
            pltpu.SemaphoreType.DMA,
            pltpu.SemaphoreType.DMA,
        ],
    )
    def k(z_hbm, sidx_hbm, didx_hbm, out_hbm,
          sidx_v, didx_v, sb0, sb1, db0, db1, out_v, tr, z_sh, sem0, sem1):
        cid = lax.axis_index("c")
        sid = lax.axis_index("s")
        wid = sid * NC + cid

        band = n_nodes // NS
        pltpu.sync_copy(z_hbm.at[pl.ds(sid * band, band)],
                        z_sh.at[pl.ds(sid * band, band)])
        pltpu.sync_copy(sidx_hbm.at[wid], sidx_v)
        pltpu.sync_copy(didx_hbm.at[wid], didx_v)
        plsc.subcore_barrier()

        sbufs = (sb0, sb1)
        dbufs = (db0, db1)
        sems = (sem0, sem1)

        def start(j, b):
            pltpu.async_copy(z_sh.at[sidx_v.at[j]], sbufs[b], sems[b])
            pltpu.async_copy(z_sh.at[didx_v.at[j]], dbufs[b], sems[b])

        def wait(b):
            pltpu.make_async_copy(z_sh.at[sidx_v.at[0]], sbufs[b], sems[b]).wait()
            pltpu.make_async_copy(z_sh.at[didx_v.at[0]], dbufs[b], sems[b]).wait()

        lanes = lax.iota(jnp.int32, 16)

        def compute(jj, b):
            sb, db = sbufs[b], dbufs[b]

            def gbody(g, carry):
                for ii in range(16):
                    e = g * 16 + ii
                    a0 = jnp.zeros((16,), jnp.float32)
                    a1 = jnp.zeros((16,), jnp.float32)
                    for kk in range(D // 32):
                        p = sb[e, pl.ds(kk * 32, 32)] * db[e, pl.ds(kk * 32, 32)]
                        p0, p1 = plsc.unpack(
                            p, format=plsc.PackFormat.INTERLEAVED,
                            preferred_element_type=jnp.float32)
                        a0 = a0 + p0
                        a1 = a1 + p1
                    tr[e, pl.ds(0, 16)] = a0 + a1
                rows = g * 16 + lanes
                cols = [plsc.load_gather(tr, [rows, jnp.full((16,), dd, jnp.int32)])
                        for dd in range(16)]
                while len(cols) > 1:
                    cols = [x + y for x, y in zip(cols[::2], cols[1::2])]
                out_v[pl.ds(jj * C + g * 16, 16)] = cols[0]
                return carry

            lax.fori_loop(0, C // 16, gbody, 0)

        start(0, 0)

        def pair(p, carry):
            j = p * 2
            for b in range(2):
                jj = j + b
                nxt = jj + 1

                @pl.when(nxt < chunks)
                def _():
                    start(nxt, (b + 1) % 2)

                wait(b)
                compute(jj, b)
            return carry

        lax.fori_loop(0, chunks // 2, pair, 0)

        def sig(i, carry):
            for u in range(2):
                o = (i * 2 + u) * 16
                x = out_v[pl.ds(o, 16)]
                out_v[pl.ds(o, 16)] = 1.0 / (1.0 + jnp.exp(-x))
            return carry

        lax.fori_loop(0, epw // 32, sig, 0)
        pltpu.sync_copy(out_v, out_hbm.at[pl.ds(wid * epw, epw)])

    return k(z, sidx, didx)


def kernel(z, edge_index):
    e = edge_index.shape[1]
    per_pair = NW * C
    chunks = -(-e // per_pair)
    chunks += chunks % 2
    epw = chunks * C
    e_pad = NW * epw
    sidx = jnp.pad(edge_index[0], (0, e_pad - e)).reshape(NW, chunks, C)
    didx = jnp.pad(edge_index[1], (0, e_pad - e)).reshape(NW, chunks, C)
    zb = z.astype(jnp.bfloat16)
    out = _sc_decode(zb, sidx, didx, chunks=chunks, epw=epw, e_pad=e_pad)
    return out[:e]
